# Initial kernel scaffold; baseline (speedup 1.0000x reference)
import jax
import jax.numpy as jnp
from jax.experimental import pallas as pl


def kernel(features_0, features_1, features_2, features_3, features_4, features_5, features_6, features_7, in_proj_W, in_proj_b, conv_W, conv_b, emb_W, emb_b, ln_g, ln_b, depth_gain):
    raise NotImplementedError("write your pallas kernel here")


# speedup vs baseline: 2.8940x; 2.8940x over previous
import functools

import numpy as np
import jax
import jax.numpy as jnp
from jax import lax
from jax.experimental import pallas as pl
from jax.experimental.pallas import tpu as pltpu
from jax.experimental.pallas import tpu_sc as plsc

MAXD = 7
H = 128
ND = [4 ** d for d in range(MAXD + 1)]
OFF = np.concatenate([[0], np.cumsum(ND)]).astype(np.int32)
TOT = int(OFF[-1])
NWORK = 32
FEAT_CHUNK = 688
TOT_PAD = FEAT_CHUNK * NWORK
TAPS = [(dy, dx) for dy in (-1, 0, 1) for dx in (-1, 0, 1)]


def _interleave(x):
    x = np.asarray(x, np.int64) & 0xFFFF
    x = (x | (x << 8)) & 0x00FF00FF
    x = (x | (x << 4)) & 0x0F0F0F0F
    x = (x | (x << 2)) & 0x33333333
    x = (x | (x << 1)) & 0x55555555
    return x


def _tables(d):
    n = 1 << d
    N = n * n
    i = np.arange(N)
    x, y = i % n, i // n
    morton_of_grid = (_interleave(x) | (_interleave(y) << 1)).astype(np.int32)
    grid_of_morton = np.empty(N, np.int32)
    grid_of_morton[morton_of_grid] = i.astype(np.int32)
    xc = (x.astype(np.float32) + 0.5) / np.float32(n)
    yc = (y.astype(np.float32) + 0.5) / np.float32(n)
    dn = np.full(N, np.float32(d) / np.float32(MAXD), np.float32)
    pos = np.stack([xc, yc, dn], 1)
    freqs = (2.0 ** np.arange(6)).astype(np.float32).reshape(1, 1, -1)
    xx = pos[..., None].astype(np.float32) * np.float32(np.pi) * 2.0 * freqs
    enc = np.concatenate([np.sin(xx), np.cos(xx)], -1).reshape(N, -1)
    posf = np.concatenate([pos, enc], 1).astype(np.float32)
    masks = np.zeros((N, 9), np.float32)
    for j, (dy, dx) in enumerate(TAPS):
        valid = (x + dx >= 0) & (x + dx < n) & (y + dy >= 0) & (y + dy < n)
        masks[:, j] = valid.astype(np.float32)
    return morton_of_grid, grid_of_morton, np.concatenate([posf, masks], 1)


_TABS = [_tables(d) for d in range(MAXD + 1)]

_FEAT_SRC = np.zeros(TOT_PAD, np.int32)
for _d in range(MAXD + 1):
    _FEAT_SRC[OFF[_d]:OFF[_d + 1]] = OFF[_d] + _TABS[_d][0]

_SX = {}
for _d in range(1, MAXD + 1):
    _n = 1 << _d
    _m = _n // 2
    _rows = max(_m, 8)
    _s = np.zeros((_rows, _n), np.float32)
    for _X in range(_m):
        _s[_X, 2 * _X] = 0.25
        _s[_X, 2 * _X + 1] = 0.25
    _SX[_d] = _s


@functools.lru_cache(maxsize=None)
def _sc_mesh():
    return plsc.VectorSubcoreMesh(core_axis_name="c", subcore_axis_name="s",
                                  num_cores=2, num_subcores=16)


def _sc_feat_body(src_hbm, idx_hbm, out_hbm, tab_v, idx_v, out_v):
    w = lax.axis_index("s") * 2 + lax.axis_index("c")
    base = w * FEAT_CHUNK
    pltpu.sync_copy(src_hbm, tab_v)
    pltpu.sync_copy(idx_hbm.at[pl.ds(base, FEAT_CHUNK)], idx_v)

    def body(i, c):
        ids = idx_v[pl.ds(i * 16, 16)]
        out_v[pl.ds(i * 16, 16)] = plsc.load_gather(tab_v, [ids])
        return c

    lax.fori_loop(0, FEAT_CHUNK // 16, body, 0)
    pltpu.sync_copy(out_v, out_hbm.at[pl.ds(base, FEAT_CHUNK)])


@functools.lru_cache(maxsize=None)
def _sc_feat_gather():
    return pl.kernel(
        _sc_feat_body,
        out_type=jax.ShapeDtypeStruct((TOT_PAD,), jnp.float32),
        mesh=_sc_mesh(),
        compiler_params=pltpu.CompilerParams(needs_layout_passes=False),
        scratch_types=[
            pltpu.VMEM((TOT_PAD,), jnp.float32),
            pltpu.VMEM((FEAT_CHUNK,), jnp.int32),
            pltpu.VMEM((FEAT_CHUNK,), jnp.float32),
        ],
    )

_SC2_DEPTHS = list(range(2, MAXD + 1))


def _sc_unshuffle_body(*refs):
    nd = len(_SC2_DEPTHS)
    e_refs = refs[0:nd]
    i_refs = refs[nd:2 * nd]
    o_refs = refs[2 * nd:3 * nd]
    idx_v, rows_v, sem = refs[3 * nd:]
    w = lax.axis_index("s") * 2 + lax.axis_index("c")

    for t, d in enumerate(_SC2_DEPTHS):
        N = ND[d]
        rw = N if N < 8 * NWORK else N // NWORK
        chunk = min(rw, 128)
        nch = rw // chunk

        def run(base, t=t, rw=rw, chunk=chunk, nch=nch):
            pltpu.sync_copy(i_refs[t].at[pl.ds(base, rw)], idx_v.at[pl.ds(0, rw)])
            for k in range(nch):
                pltpu.async_copy(
                    e_refs[t].at[idx_v.at[pl.ds(k * chunk, chunk)]],
                    rows_v.at[pl.ds(0, chunk)], sem).wait()
                pltpu.sync_copy(rows_v.at[pl.ds(0, chunk)],
                                o_refs[t].at[pl.ds(base + k * chunk, chunk)])

        if N < 8 * NWORK:
            @pl.when(w == 0)
            def _():
                run(0)
        else:
            run(w * rw)


@functools.lru_cache(maxsize=None)
def _sc_unshuffle():
    return pl.kernel(
        _sc_unshuffle_body,
        out_type=tuple(jax.ShapeDtypeStruct((ND[d], H), jnp.float32)
                       for d in _SC2_DEPTHS),
        mesh=_sc_mesh(),
        scratch_types=[
            pltpu.VMEM((512,), jnp.int32),
            pltpu.VMEM((128, H), jnp.float32),
            pltpu.SemaphoreType.DMA,
        ],
    )


def _shift_up(T, s, N):
    k = s % N
    if k == 0:
        return T
    return jnp.concatenate([T[k:], T[:k]], axis=0)


def _tc_body(*refs):
    it = iter(refs)
    X = [next(it) for _ in range(8)]
    inW = next(it)
    inb = next(it)
    Wcat = {d: next(it) for d in range(1, 7)}
    convb = {d: next(it) for d in range(1, 7)}
    embW = [next(it) for _ in range(8)]
    embb = [next(it) for _ in range(8)]
    g2 = [next(it) for _ in range(8)]
    b2 = [next(it) for _ in range(8)]
    Sx = {d: next(it) for d in range(1, 8)}
    E = [next(it) for _ in range(8)]

    Wv = inW[...]
    bv = inb[...]
    for d in range(8):
        A = X[d][...][:, 0:40]
        E[d][...] = jnp.dot(A, Wv, preferred_element_type=jnp.float32) + bv

    for d in range(7, 0, -1):
        n = 1 << d
        m = n // 2
        Sxv = Sx[d][...]
        for Y in range(m):
            rA = E[d][pl.ds((2 * Y) * n, n), :]
            rB = E[d][pl.ds((2 * Y + 1) * n, n), :]
            ch = jnp.dot(Sxv, rA + rB, preferred_element_type=jnp.float32)
            E[d - 1][pl.ds(Y * m, m), :] = E[d - 1][pl.ds(Y * m, m), :] + ch[:m]
        dc = d - 1
        if dc >= 1:
            nc = 1 << dc
            Nc = ND[dc]
            hv = E[dc][...]
            Wc = Wcat[dc][...]
            Xm = X[dc][:, 40:49]
            acc = convb[dc][...]
            for j, (dy, dx) in enumerate(TAPS):
                T = jnp.dot(hv, Wc[:, j * H:(j + 1) * H],
                            preferred_element_type=jnp.float32)
                T = _shift_up(T, dy * nc + dx, Nc)
                acc = acc + Xm[:, j:j + 1] * T
            E[dc][...] = jnp.maximum(acc, 0.0)

    for d in range(8):
        hv = E[d][...]
        z = jnp.dot(hv, embW[d][...], preferred_element_type=jnp.float32) + embb[d][...]
        mu = jnp.mean(z, axis=1, keepdims=True)
        zc = z - mu
        var = jnp.mean(zc * zc, axis=1, keepdims=True)
        zn = zc * lax.rsqrt(var + 1e-5)
        E[d][...] = zn * g2[d][...] + b2[d][...]


_tc_pyramid = pl.pallas_call(
    _tc_body,
    out_shape=tuple(jax.ShapeDtypeStruct((ND[d], H), jnp.float32)
                    for d in range(8)),
)


def kernel(features_0, features_1, features_2, features_3, features_4,
           features_5, features_6, features_7, in_proj_W, in_proj_b,
           conv_W, conv_b, emb_W, emb_b, ln_g, ln_b, depth_gain):
    feats = [features_0, features_1, features_2, features_3, features_4,
             features_5, features_6, features_7]
    f32 = jnp.float32

    feats_all = jnp.concatenate(
        [f.reshape(-1) for f in feats]
        + [jnp.zeros((TOT_PAD - TOT,), f32)])
    fg = _sc_feat_gather()(feats_all, jnp.asarray(_FEAT_SRC))

    ops = []
    for d in range(8):
        fcol = fg[OFF[d]:OFF[d] + ND[d]].reshape(ND[d], 1)
        ops.append(jnp.concatenate([fcol, jnp.asarray(_TABS[d][2])], axis=1))
    ops.append(in_proj_W)
    ops.append(in_proj_b.reshape(1, H))
    for d in range(1, 7):
        ops.append(conv_W[d].reshape(9, H, H).transpose(1, 0, 2).reshape(H, 9 * H))
    for d in range(1, 7):
        ops.append(conv_b[d].reshape(1, H))
    for d in range(8):
        ops.append(emb_W[d])
    for d in range(8):
        ops.append(emb_b[d].reshape(1, H))
    for d in range(8):
        ops.append((depth_gain[d] * ln_g[d]).reshape(1, H))
    for d in range(8):
        ops.append((depth_gain[d] * ln_b[d]).reshape(1, H))
    for d in range(1, 8):
        ops.append(jnp.asarray(_SX[d]))

    Eg = _tc_pyramid(*ops)

    Em = _sc_unshuffle()(
        *[Eg[d] for d in _SC2_DEPTHS],
        *[jnp.asarray(_TABS[d][1]) for d in _SC2_DEPTHS])

    return (Eg[0], Eg[1]) + tuple(Em)
